# baseline (speedup 1.0000x reference)
import functools

import jax
import jax.numpy as jnp
from jax import lax
from jax.experimental import pallas as pl
from jax.experimental.pallas import tpu as pltpu
from jax.experimental.pallas import tpu_sc as plsc

N_NODES = 10000
N_EDGES = 320000
D_IN = 128
D_HID = 64
D_OUT = 32

NW = 32
E_PER_W = N_EDGES // NW
EPB = 125
BK = 128
NB = E_PER_W // EPB
N_PAD = 10240
ROWS_PER_TILE = N_PAD // 16


def _make_spmm(D):
    nfv = D // 16
    mesh = plsc.VectorSubcoreMesh(core_axis_name="c", subcore_axis_name="s")

    @functools.partial(
        pl.kernel,
        mesh=mesh,
        compiler_params=pltpu.CompilerParams(use_tc_tiling_on_sc=False),
        out_type=jax.ShapeDtypeStruct((2, N_PAD, D), jnp.float32),
        scratch_types=[
            pltpu.VMEM((NB, BK), jnp.int32),
            pltpu.VMEM((NB, BK), jnp.int32),
            pltpu.VMEM((NB, BK), jnp.float32),
            pltpu.VMEM((BK, D), jnp.float32),
            pltpu.VMEM((BK, D), jnp.float32),
            pltpu.VMEM((BK, D), jnp.float32),
            pltpu.VMEM((BK, D), jnp.float32),
            pltpu.VMEM((BK, D), jnp.float32),
            pltpu.VMEM_SHARED((N_PAD, D), jnp.float32),
            pltpu.SemaphoreType.DMA,
            pltpu.SemaphoreType.DMA,
            pltpu.SemaphoreType.DMA,
            pltpu.SemaphoreType.DMA,
            pltpu.SemaphoreType.DMA,
            pltpu.SemaphoreType.DMA,
            pltpu.SemaphoreType.DMA,
            pltpu.SemaphoreType.DMA,
            pltpu.SemaphoreType.DMA,
            pltpu.SemaphoreType.DMA,
        ],
    )
    def spmm(src_hbm, dst_hbm, wgt_hbm, h_hbm, zeros_hbm, out_hbm,
             src_v, dst_v, wgt_v, rows0, rows1, rows2, rows3, rows4, accum,
             gsem0, gsem1, gsem2, gsem3, gsem4,
             ssem0, ssem1, ssem2, ssem3, ssem4):
        c = lax.axis_index("c")
        s = lax.axis_index("s")
        wid = c * 16 + s
        pltpu.sync_copy(src_hbm.at[wid], src_v)
        pltpu.sync_copy(dst_hbm.at[wid], dst_v)
        pltpu.sync_copy(wgt_hbm.at[wid], wgt_v)
        pltpu.sync_copy(zeros_hbm.at[pl.ds(s * ROWS_PER_TILE, ROWS_PER_TILE)],
                        accum.at[pl.ds(s * ROWS_PER_TILE, ROWS_PER_TILE)])
        plsc.subcore_barrier()

        def scale(buf, b):
            for g in range(BK // 16):
                wv = wgt_v[b, pl.ds(g * 16, 16)]
                for e in range(16):
                    sc = lax.gather(
                        wv, jnp.full((16, 1), e, jnp.int32),
                        lax.GatherDimensionNumbers(
                            offset_dims=(), collapsed_slice_dims=(0,),
                            start_index_map=(0,)),
                        slice_sizes=(1,),
                        mode=lax.GatherScatterMode.PROMISE_IN_BOUNDS)
                    r = g * 16 + e
                    for f in range(nfv):
                        buf[r, pl.ds(f * 16, 16)] = (
                            buf[r, pl.ds(f * 16, 16)] * sc)

        bufs = (rows0, rows1, rows2, rows3, rows4)
        gsems = (gsem0, gsem1, gsem2, gsem3, gsem4)
        ssems = (ssem0, ssem1, ssem2, ssem3, ssem4)
        R = 5

        def issue_gather(b, buf, sem):
            pltpu.async_copy(h_hbm.at[src_v.at[b]], buf, sem)

        def wait_gather(b, buf, sem):
            pltpu.make_async_copy(h_hbm.at[src_v.at[b]], buf, sem).wait()

        def issue_scatter(b, buf, sem):
            pltpu.async_copy(buf, accum.at[dst_v.at[b]], sem, add=True)

        def wait_scatter(b, buf, sem):
            pltpu.make_async_copy(buf, accum.at[dst_v.at[b]], sem).wait()

        for k in range(R):
            issue_gather(k, bufs[k], gsems[k])

        @pl.loop(0, NB, step=R)
        def _(b):
            for k in range(R):
                t = b + k
                kp = (k - 1) % R
                wait_gather(t, bufs[k], gsems[k])
                scale(bufs[k], t)
                issue_scatter(t, bufs[k], ssems[k])

                @pl.when(t > 0)
                def _():
                    wait_scatter(t - 1, bufs[kp], ssems[kp])

                    @pl.when(t - 1 + R < NB)
                    def _():
                        issue_gather(t - 1 + R, bufs[kp], gsems[kp])

        wait_scatter(NB - 1, bufs[(NB - 1) % R], ssems[(NB - 1) % R])

        plsc.subcore_barrier()
        pltpu.sync_copy(accum.at[pl.ds(s * ROWS_PER_TILE, ROWS_PER_TILE)],
                        out_hbm.at[c, pl.ds(s * ROWS_PER_TILE, ROWS_PER_TILE)])

    return spmm


_spmm64 = _make_spmm(D_HID)
_spmm32 = _make_spmm(D_OUT)


def _mm_body(x_ref, w_ref, o_ref):
    o_ref[...] = jnp.dot(x_ref[...], w_ref[...],
                         preferred_element_type=jnp.float32)


def _mm(x, w, bm):
    m, k = x.shape
    _, n = w.shape
    return pl.pallas_call(
        _mm_body,
        grid=(m // bm,),
        in_specs=[pl.BlockSpec((bm, k), lambda i: (i, 0)),
                  pl.BlockSpec((k, n), lambda i: (0, 0))],
        out_specs=pl.BlockSpec((bm, n), lambda i: (i, 0)),
        out_shape=jax.ShapeDtypeStruct((m, n), jnp.float32),
    )(x, w)


def _l2_body(p0_ref, p1_ref, w_ref, o_ref):
    h = jnp.maximum(p0_ref[0] + p1_ref[0], 0.0)
    o_ref[...] = jnp.dot(h, w_ref[...], preferred_element_type=jnp.float32)


def _l2(p, w, bm, m):
    k = p.shape[2]
    n = w.shape[1]
    return pl.pallas_call(
        _l2_body,
        grid=(m // bm,),
        in_specs=[pl.BlockSpec((1, bm, k), lambda i: (0, i, 0)),
                  pl.BlockSpec((1, bm, k), lambda i: (1, i, 0)),
                  pl.BlockSpec((k, n), lambda i: (0, 0))],
        out_specs=pl.BlockSpec((bm, n), lambda i: (i, 0)),
        out_shape=jax.ShapeDtypeStruct((m, n), jnp.float32),
    )(p, p, w)


def _dec_body(p0i_ref, p1i_ref, p0j_ref, p1j_ref, recon_ref, emb_ref):
    zi = p0i_ref[0] + p1i_ref[0]
    zj = p0j_ref[0] + p1j_ref[0]
    recon_ref[...] = lax.dot_general(zi, zj, (((1,), (1,)), ((), ())),
                                     preferred_element_type=jnp.float32)

    @pl.when(pl.program_id(1) == 0)
    def _():
        emb_ref[...] = zi


def _decoder(q, bm, bn, m):
    k = q.shape[2]
    nbi = m // bm
    nbj = pl.cdiv(m, bn)
    return pl.pallas_call(
        _dec_body,
        grid=(nbi, nbj),
        in_specs=[pl.BlockSpec((1, bm, k), lambda i, j: (0, i, 0)),
                  pl.BlockSpec((1, bm, k), lambda i, j: (1, i, 0)),
                  pl.BlockSpec((1, bn, k), lambda i, j: (0, j, 0)),
                  pl.BlockSpec((1, bn, k), lambda i, j: (1, j, 0))],
        out_specs=[pl.BlockSpec((bm, bn), lambda i, j: (i, j)),
                   pl.BlockSpec((bm, k), lambda i, j: (i, 0))],
        out_shape=[jax.ShapeDtypeStruct((m, m), jnp.float32),
                   jax.ShapeDtypeStruct((m, k), jnp.float32)],
    )(q, q, q, q)


def kernel(x, edge_index, edge_weight, W0, W1):
    pad = ((0, 0), (0, 0), (0, BK - EPB))
    src = jnp.pad(edge_index[0].astype(jnp.int32).reshape(NW, NB, EPB), pad)
    dst = jnp.pad(edge_index[1].astype(jnp.int32).reshape(NW, NB, EPB), pad)
    wgt = jnp.pad(edge_weight.reshape(NW, NB, EPB), pad)
    zeros64 = jnp.zeros((N_PAD, D_HID), jnp.float32)
    zeros32 = jnp.zeros((N_PAD, D_OUT), jnp.float32)

    h0 = _mm(x, W0, 1000)
    p = _spmm64(src, dst, wgt, h0, zeros64)
    h1 = _l2(p, W1, 1000, N_NODES)
    q = _spmm32(src, dst, wgt, h1, zeros32)
    recon, emb = _decoder(q, 2000, 2048, N_NODES)
    return (recon, emb)


# speedup vs baseline: 1.5503x; 1.5503x over previous
import functools

import jax
import jax.numpy as jnp
from jax import lax
from jax.experimental import pallas as pl
from jax.experimental.pallas import tpu as pltpu
from jax.experimental.pallas import tpu_sc as plsc

N_NODES = 10000
N_EDGES = 320000
D_IN = 128
D_HID = 64
D_OUT = 32

NW = 32
E_PER_W = N_EDGES // NW
BK = 80
NB = E_PER_W // BK
N_PAD = 10240
ROWS_PER_TILE = N_PAD // 16


def _make_spmm(D):
    nfv = D // 16
    mesh = plsc.VectorSubcoreMesh(core_axis_name="c", subcore_axis_name="s")

    @functools.partial(
        pl.kernel,
        mesh=mesh,
        compiler_params=pltpu.CompilerParams(use_tc_tiling_on_sc=False),
        out_type=jax.ShapeDtypeStruct((2, N_PAD, D), jnp.float32),
        scratch_types=[
            pltpu.VMEM((NB, BK), jnp.int32),
            pltpu.VMEM((NB, BK), jnp.int32),
            pltpu.VMEM((NB, BK), jnp.float32),
            pltpu.VMEM((BK, D), jnp.float32),
            pltpu.VMEM((BK, D), jnp.float32),
            pltpu.VMEM((BK, D), jnp.float32),
            pltpu.VMEM((BK, D), jnp.float32),
            pltpu.VMEM((BK, D), jnp.float32),
            pltpu.VMEM_SHARED((N_PAD, D), jnp.float32),
            pltpu.SemaphoreType.DMA,
            pltpu.SemaphoreType.DMA,
            pltpu.SemaphoreType.DMA,
            pltpu.SemaphoreType.DMA,
            pltpu.SemaphoreType.DMA,
            pltpu.SemaphoreType.DMA,
            pltpu.SemaphoreType.DMA,
            pltpu.SemaphoreType.DMA,
            pltpu.SemaphoreType.DMA,
            pltpu.SemaphoreType.DMA,
        ],
    )
    def spmm(src_hbm, dst_hbm, wgt_hbm, h_hbm, zeros_hbm, out_hbm,
             src_v, dst_v, wgt_v, rows0, rows1, rows2, rows3, rows4, accum,
             gsem0, gsem1, gsem2, gsem3, gsem4,
             ssem0, ssem1, ssem2, ssem3, ssem4):
        c = lax.axis_index("c")
        s = lax.axis_index("s")
        wid = c * 16 + s
        pltpu.sync_copy(src_hbm.at[wid], src_v)
        pltpu.sync_copy(dst_hbm.at[wid], dst_v)
        pltpu.sync_copy(wgt_hbm.at[wid], wgt_v)
        pltpu.sync_copy(zeros_hbm.at[pl.ds(s * ROWS_PER_TILE, ROWS_PER_TILE)],
                        accum.at[pl.ds(s * ROWS_PER_TILE, ROWS_PER_TILE)])
        plsc.subcore_barrier()

        def scale(buf, b):
            for g in range(BK // 16):
                wv = wgt_v[b, pl.ds(g * 16, 16)]
                for e in range(16):
                    sc = lax.gather(
                        wv, jnp.full((16, 1), e, jnp.int32),
                        lax.GatherDimensionNumbers(
                            offset_dims=(), collapsed_slice_dims=(0,),
                            start_index_map=(0,)),
                        slice_sizes=(1,),
                        mode=lax.GatherScatterMode.PROMISE_IN_BOUNDS)
                    r = g * 16 + e
                    for f in range(nfv):
                        buf[r, pl.ds(f * 16, 16)] = (
                            buf[r, pl.ds(f * 16, 16)] * sc)

        bufs = (rows0, rows1, rows2, rows3, rows4)
        gsems = (gsem0, gsem1, gsem2, gsem3, gsem4)
        ssems = (ssem0, ssem1, ssem2, ssem3, ssem4)
        R = 5

        def issue_gather(b, buf, sem):
            pltpu.async_copy(h_hbm.at[src_v.at[b]], buf, sem)

        def wait_gather(b, buf, sem):
            pltpu.make_async_copy(h_hbm.at[src_v.at[b]], buf, sem).wait()

        def issue_scatter(b, buf, sem):
            pltpu.async_copy(buf, accum.at[dst_v.at[b]], sem, add=True)

        def wait_scatter(b, buf, sem):
            pltpu.make_async_copy(buf, accum.at[dst_v.at[b]], sem).wait()

        for k in range(R):
            issue_gather(k, bufs[k], gsems[k])

        @pl.loop(0, NB, step=R)
        def _(b):
            for k in range(R):
                t = b + k
                kp = (k - 1) % R
                wait_gather(t, bufs[k], gsems[k])
                scale(bufs[k], t)
                issue_scatter(t, bufs[k], ssems[k])

                @pl.when(t > 0)
                def _():
                    wait_scatter(t - 1, bufs[kp], ssems[kp])

                    @pl.when(t - 1 + R < NB)
                    def _():
                        issue_gather(t - 1 + R, bufs[kp], gsems[kp])

        wait_scatter(NB - 1, bufs[(NB - 1) % R], ssems[(NB - 1) % R])

        plsc.subcore_barrier()
        pltpu.sync_copy(accum.at[pl.ds(s * ROWS_PER_TILE, ROWS_PER_TILE)],
                        out_hbm.at[c, pl.ds(s * ROWS_PER_TILE, ROWS_PER_TILE)])

    return spmm


_spmm64 = _make_spmm(D_HID)
_spmm32 = _make_spmm(D_OUT)


def _mm_body(x_ref, w_ref, o_ref):
    o_ref[...] = jnp.dot(x_ref[...], w_ref[...],
                         preferred_element_type=jnp.float32)


def _mm(x, w, bm):
    m, k = x.shape
    _, n = w.shape
    return pl.pallas_call(
        _mm_body,
        grid=(m // bm,),
        in_specs=[pl.BlockSpec((bm, k), lambda i: (i, 0)),
                  pl.BlockSpec((k, n), lambda i: (0, 0))],
        out_specs=pl.BlockSpec((bm, n), lambda i: (i, 0)),
        out_shape=jax.ShapeDtypeStruct((m, n), jnp.float32),
    )(x, w)


def _l2_body(p0_ref, p1_ref, w_ref, o_ref):
    h = jnp.maximum(p0_ref[0] + p1_ref[0], 0.0)
    o_ref[...] = jnp.dot(h, w_ref[...], preferred_element_type=jnp.float32)


def _l2(p, w, bm, m):
    k = p.shape[2]
    n = w.shape[1]
    return pl.pallas_call(
        _l2_body,
        grid=(m // bm,),
        in_specs=[pl.BlockSpec((1, bm, k), lambda i: (0, i, 0)),
                  pl.BlockSpec((1, bm, k), lambda i: (1, i, 0)),
                  pl.BlockSpec((k, n), lambda i: (0, 0))],
        out_specs=pl.BlockSpec((bm, n), lambda i: (i, 0)),
        out_shape=jax.ShapeDtypeStruct((m, n), jnp.float32),
    )(p, p, w)


def _dec_body(p0i_ref, p1i_ref, p0j_ref, p1j_ref, recon_ref, emb_ref):
    zi = p0i_ref[0] + p1i_ref[0]
    zj = p0j_ref[0] + p1j_ref[0]
    recon_ref[...] = lax.dot_general(zi, zj, (((1,), (1,)), ((), ())),
                                     preferred_element_type=jnp.float32)

    @pl.when(pl.program_id(1) == 0)
    def _():
        emb_ref[...] = zi


def _decoder(q, bm, bn, m):
    k = q.shape[2]
    nbi = m // bm
    nbj = pl.cdiv(m, bn)
    return pl.pallas_call(
        _dec_body,
        grid=(nbi, nbj),
        in_specs=[pl.BlockSpec((1, bm, k), lambda i, j: (0, i, 0)),
                  pl.BlockSpec((1, bm, k), lambda i, j: (1, i, 0)),
                  pl.BlockSpec((1, bn, k), lambda i, j: (0, j, 0)),
                  pl.BlockSpec((1, bn, k), lambda i, j: (1, j, 0))],
        out_specs=[pl.BlockSpec((bm, bn), lambda i, j: (i, j)),
                   pl.BlockSpec((bm, k), lambda i, j: (i, 0))],
        out_shape=[jax.ShapeDtypeStruct((m, m), jnp.float32),
                   jax.ShapeDtypeStruct((m, k), jnp.float32)],
    )(q, q, q, q)


def kernel(x, edge_index, edge_weight, W0, W1):
    src = edge_index[0].astype(jnp.int32).reshape(NW, NB, BK)
    dst = edge_index[1].astype(jnp.int32).reshape(NW, NB, BK)
    wgt = edge_weight.reshape(NW, NB, BK)
    zeros64 = jnp.zeros((N_PAD, D_HID), jnp.float32)
    zeros32 = jnp.zeros((N_PAD, D_OUT), jnp.float32)

    h0 = _mm(x, W0, 1000)
    p = _spmm64(src, dst, wgt, h0, zeros64)
    h1 = _l2(p, W1, 1000, N_NODES)
    q = _spmm32(src, dst, wgt, h1, zeros32)
    recon, emb = _decoder(q, 2000, 2048, N_NODES)
    return (recon, emb)


# speedup vs baseline: 1.5853x; 1.0226x over previous
import functools

import jax
import jax.numpy as jnp
from jax import lax
from jax.experimental import pallas as pl
from jax.experimental.pallas import tpu as pltpu
from jax.experimental.pallas import tpu_sc as plsc

N_NODES = 10000
N_EDGES = 320000
D_IN = 128
D_HID = 64
D_OUT = 32

NW = 32
E_PER_W = N_EDGES // NW
BK = 80
NB = E_PER_W // BK
N_PAD = 10240
ROWS_PER_TILE = N_PAD // 16


DP = 32


def _make_spmm(D):
    npass = D // DP
    nfv = DP // 16
    mesh = plsc.VectorSubcoreMesh(core_axis_name="c", subcore_axis_name="s")

    @functools.partial(
        pl.kernel,
        mesh=mesh,
        compiler_params=pltpu.CompilerParams(use_tc_tiling_on_sc=False),
        out_type=jax.ShapeDtypeStruct((2, N_PAD, D), jnp.float32),
        scratch_types=[
            pltpu.VMEM((NB, BK), jnp.int32),
            pltpu.VMEM((NB, BK), jnp.int32),
            pltpu.VMEM((NB, BK), jnp.float32),
            pltpu.VMEM((BK, DP), jnp.float32),
            pltpu.VMEM((BK, DP), jnp.float32),
            pltpu.VMEM((BK, DP), jnp.float32),
            pltpu.VMEM((BK, DP), jnp.float32),
            pltpu.VMEM((BK, DP), jnp.float32),
            pltpu.VMEM_SHARED((N_PAD, DP), jnp.float32),
            pltpu.VMEM_SHARED((N_PAD, DP), jnp.float32),
            pltpu.SemaphoreType.DMA,
            pltpu.SemaphoreType.DMA,
            pltpu.SemaphoreType.DMA,
            pltpu.SemaphoreType.DMA,
            pltpu.SemaphoreType.DMA,
            pltpu.SemaphoreType.DMA,
            pltpu.SemaphoreType.DMA,
            pltpu.SemaphoreType.DMA,
            pltpu.SemaphoreType.DMA,
            pltpu.SemaphoreType.DMA,
        ],
    )
    def spmm(src_hbm, dst_hbm, wgt_hbm, h_hbm, zeros_hbm, out_hbm,
             src_v, dst_v, wgt_v, rows0, rows1, rows2, rows3, rows4, accum,
             h_loc,
             gsem0, gsem1, gsem2, gsem3, gsem4,
             ssem0, ssem1, ssem2, ssem3, ssem4):
        c = lax.axis_index("c")
        s = lax.axis_index("s")
        wid = c * 16 + s
        row0 = s * ROWS_PER_TILE

        pltpu.async_copy(src_hbm.at[wid], src_v, gsem0)
        pltpu.async_copy(dst_hbm.at[wid], dst_v, gsem1)
        pltpu.async_copy(wgt_hbm.at[wid], wgt_v, gsem2)

        def zero_and_stage(p):
            pltpu.async_copy(zeros_hbm.at[pl.ds(row0, ROWS_PER_TILE)],
                             accum.at[pl.ds(row0, ROWS_PER_TILE)], gsem3)

            @pl.when(s < 15)
            def _():
                pltpu.async_copy(
                    h_hbm.at[pl.ds(row0, ROWS_PER_TILE), pl.ds(p * DP, DP)],
                    h_loc.at[pl.ds(row0, ROWS_PER_TILE)], ssem0)
                pltpu.make_async_copy(
                    h_hbm.at[pl.ds(row0, ROWS_PER_TILE), pl.ds(p * DP, DP)],
                    h_loc.at[pl.ds(row0, ROWS_PER_TILE)], ssem0).wait()

            @pl.when(s == 15)
            def _():
                tail = N_NODES - 15 * ROWS_PER_TILE
                pltpu.async_copy(
                    h_hbm.at[pl.ds(15 * ROWS_PER_TILE, tail),
                             pl.ds(p * DP, DP)],
                    h_loc.at[pl.ds(15 * ROWS_PER_TILE, tail)], ssem0)
                pltpu.make_async_copy(
                    h_hbm.at[pl.ds(15 * ROWS_PER_TILE, tail),
                             pl.ds(p * DP, DP)],
                    h_loc.at[pl.ds(15 * ROWS_PER_TILE, tail)], ssem0).wait()

            pltpu.make_async_copy(
                zeros_hbm.at[pl.ds(row0, ROWS_PER_TILE)],
                accum.at[pl.ds(row0, ROWS_PER_TILE)], gsem3).wait()

        def scale(buf, b):
            for g in range(BK // 16):
                wv = wgt_v[b, pl.ds(g * 16, 16)]
                for e in range(16):
                    w = wv[e]
                    r = g * 16 + e
                    for f in range(nfv):
                        buf[r, pl.ds(f * 16, 16)] = (
                            buf[r, pl.ds(f * 16, 16)] * w)

        bufs = (rows0, rows1, rows2, rows3, rows4)
        gsems = (gsem0, gsem1, gsem2, gsem3, gsem4)
        ssems = (ssem0, ssem1, ssem2, ssem3, ssem4)
        R = 5

        def issue_gather(b, buf, sem):
            pltpu.async_copy(h_loc.at[src_v.at[b]], buf, sem)

        def wait_gather(b, buf, sem):
            pltpu.make_async_copy(h_loc.at[src_v.at[b]], buf, sem).wait()

        def issue_scatter(b, buf, sem):
            pltpu.async_copy(buf, accum.at[dst_v.at[b]], sem, add=True)

        def wait_scatter(b, buf, sem):
            pltpu.make_async_copy(buf, accum.at[dst_v.at[b]], sem).wait()

        zero_and_stage(0)
        pltpu.make_async_copy(src_hbm.at[wid], src_v, gsem0).wait()
        pltpu.make_async_copy(dst_hbm.at[wid], dst_v, gsem1).wait()
        pltpu.make_async_copy(wgt_hbm.at[wid], wgt_v, gsem2).wait()

        for p in range(npass):
            plsc.subcore_barrier()

            for k in range(R):
                issue_gather(k, bufs[k], gsems[k])

            @pl.loop(0, NB, step=R)
            def _(b):
                for k in range(R):
                    t = b + k
                    kp = (k - 1) % R
                    wait_gather(t, bufs[k], gsems[k])
                    scale(bufs[k], t)
                    issue_scatter(t, bufs[k], ssems[k])

                    @pl.when(t > 0)
                    def _():
                        wait_scatter(t - 1, bufs[kp], ssems[kp])

                        @pl.when(t - 1 + R < NB)
                        def _():
                            issue_gather(t - 1 + R, bufs[kp], gsems[kp])

            wait_scatter(NB - 1, bufs[(NB - 1) % R], ssems[(NB - 1) % R])

            plsc.subcore_barrier()
            pltpu.sync_copy(accum.at[pl.ds(row0, ROWS_PER_TILE)],
                            out_hbm.at[c, pl.ds(row0, ROWS_PER_TILE),
                                       pl.ds(p * DP, DP)])
            if p + 1 < npass:
                zero_and_stage(p + 1)

    return spmm


_spmm64 = _make_spmm(D_HID)
_spmm32 = _make_spmm(D_OUT)


def _mm_body(x_ref, w_ref, o_ref):
    o_ref[...] = jnp.dot(x_ref[...], w_ref[...],
                         preferred_element_type=jnp.float32)


def _mm(x, w, bm):
    m, k = x.shape
    _, n = w.shape
    return pl.pallas_call(
        _mm_body,
        grid=(m // bm,),
        in_specs=[pl.BlockSpec((bm, k), lambda i: (i, 0)),
                  pl.BlockSpec((k, n), lambda i: (0, 0))],
        out_specs=pl.BlockSpec((bm, n), lambda i: (i, 0)),
        out_shape=jax.ShapeDtypeStruct((m, n), jnp.float32),
    )(x, w)


def _l2_body(p0_ref, p1_ref, w_ref, o_ref):
    h = jnp.maximum(p0_ref[0] + p1_ref[0], 0.0)
    o_ref[...] = jnp.dot(h, w_ref[...], preferred_element_type=jnp.float32)


def _l2(p, w, bm, m):
    k = p.shape[2]
    n = w.shape[1]
    return pl.pallas_call(
        _l2_body,
        grid=(m // bm,),
        in_specs=[pl.BlockSpec((1, bm, k), lambda i: (0, i, 0)),
                  pl.BlockSpec((1, bm, k), lambda i: (1, i, 0)),
                  pl.BlockSpec((k, n), lambda i: (0, 0))],
        out_specs=pl.BlockSpec((bm, n), lambda i: (i, 0)),
        out_shape=jax.ShapeDtypeStruct((m, n), jnp.float32),
    )(p, p, w)


def _dec_body(p0i_ref, p1i_ref, p0j_ref, p1j_ref, recon_ref, emb_ref):
    zi = p0i_ref[0] + p1i_ref[0]
    zj = p0j_ref[0] + p1j_ref[0]
    recon_ref[...] = lax.dot_general(zi, zj, (((1,), (1,)), ((), ())),
                                     preferred_element_type=jnp.float32)

    @pl.when(pl.program_id(1) == 0)
    def _():
        emb_ref[...] = zi


def _decoder(q, bm, bn, m):
    k = q.shape[2]
    nbi = m // bm
    nbj = pl.cdiv(m, bn)
    return pl.pallas_call(
        _dec_body,
        grid=(nbi, nbj),
        in_specs=[pl.BlockSpec((1, bm, k), lambda i, j: (0, i, 0)),
                  pl.BlockSpec((1, bm, k), lambda i, j: (1, i, 0)),
                  pl.BlockSpec((1, bn, k), lambda i, j: (0, j, 0)),
                  pl.BlockSpec((1, bn, k), lambda i, j: (1, j, 0))],
        out_specs=[pl.BlockSpec((bm, bn), lambda i, j: (i, j)),
                   pl.BlockSpec((bm, k), lambda i, j: (i, 0))],
        out_shape=[jax.ShapeDtypeStruct((m, m), jnp.float32),
                   jax.ShapeDtypeStruct((m, k), jnp.float32)],
    )(q, q, q, q)


def kernel(x, edge_index, edge_weight, W0, W1):
    src = edge_index[0].astype(jnp.int32).reshape(NW, NB, BK)
    dst = edge_index[1].astype(jnp.int32).reshape(NW, NB, BK)
    wgt = edge_weight.reshape(NW, NB, BK)
    zeros32 = jnp.zeros((N_PAD, DP), jnp.float32)

    h0 = _mm(x, W0, 1000)
    p = _spmm64(src, dst, wgt, h0, zeros32)
    h1 = _l2(p, W1, 1000, N_NODES)
    q = _spmm32(src, dst, wgt, h1, zeros32)
    recon, emb = _decoder(q, 2000, 2048, N_NODES)
    return (recon, emb)
